# baseline (speedup 1.0000x reference)
import functools

import jax
import jax.numpy as jnp
from jax import lax
from jax.experimental import pallas as pl
from jax.experimental.pallas import tpu as pltpu
from jax.experimental.pallas import tpu_sc as plsc

_HIDDEN = 128
_NC = 2
_NS = 16
_NW = _NC * _NS
_CHUNK = 128
_N_CHUNKS = 25
_B_PER_W = _N_CHUNKS * _CHUNK
_B_PAD = _NW * _B_PER_W


def _body(idx_hbm, table_hbm, out_hbm, idx_v, rows0, rows1, sem0, sem1):
    wid = lax.axis_index("s") * _NC + lax.axis_index("c")
    base = wid * _B_PER_W
    pltpu.sync_copy(idx_hbm.at[wid], idx_v)
    rows = (rows0, rows1)
    sems = (sem0, sem1)
    copies = [None, None]
    for c in range(_N_CHUNKS):
        b = c & 1
        copies[b] = pltpu.async_copy(table_hbm.at[idx_v.at[c]], rows[b], sems[b])
        if c > 0:
            copies[1 - b].wait()
            pltpu.sync_copy(
                rows[1 - b], out_hbm.at[pl.ds(base + (c - 1) * _CHUNK, _CHUNK)]
            )
    last = (_N_CHUNKS - 1) & 1
    copies[last].wait()
    pltpu.sync_copy(
        rows[last], out_hbm.at[pl.ds(base + (_N_CHUNKS - 1) * _CHUNK, _CHUNK)]
    )


@jax.jit
def _lookup(idx, table):
    mesh = plsc.VectorSubcoreMesh(
        core_axis_name="c", subcore_axis_name="s", num_cores=_NC, num_subcores=_NS
    )
    run = functools.partial(
        pl.kernel,
        out_type=jax.ShapeDtypeStruct((_B_PAD, _HIDDEN), jnp.float32),
        mesh=mesh,
        scratch_types=[
            pltpu.VMEM((_N_CHUNKS, _CHUNK), jnp.int32),
            pltpu.VMEM((_CHUNK, _HIDDEN), jnp.float32),
            pltpu.VMEM((_CHUNK, _HIDDEN), jnp.float32),
            pltpu.SemaphoreType.DMA,
            pltpu.SemaphoreType.DMA,
        ],
    )(_body)
    return run(idx, table)


def kernel(x, emb_weight):
    n = x.shape[0]
    x = x.astype(jnp.int32)
    x_pad = jnp.pad(x, (0, _B_PAD - n)).reshape(_NW, _N_CHUNKS, _CHUNK)
    out = _lookup(x_pad, emb_weight.astype(jnp.float32))
    return out[:n]


# speedup vs baseline: 1.0042x; 1.0042x over previous
import functools

import jax
import jax.numpy as jnp
from jax import lax
from jax.experimental import pallas as pl
from jax.experimental.pallas import tpu as pltpu
from jax.experimental.pallas import tpu_sc as plsc

_HIDDEN = 128
_NC = 2
_NS = 16
_NW = _NC * _NS
_CHUNK = 128
_N_CHUNKS = 25
_B_PER_W = _N_CHUNKS * _CHUNK
_B_PAD = _NW * _B_PER_W


_NBUF = 4


def _body(idx_hbm, table_hbm, out_hbm, idx_v, *bufs):
    rows = bufs[:_NBUF]
    gsems = bufs[_NBUF : 2 * _NBUF]
    wsems = bufs[2 * _NBUF :]
    wid = lax.axis_index("s") * _NC + lax.axis_index("c")
    base = wid * _B_PER_W
    pltpu.sync_copy(idx_hbm.at[wid], idx_v)
    gc = [None] * _NBUF
    wc = [None] * _NBUF
    for c in range(_N_CHUNKS + _NBUF - 1):
        if c < _N_CHUNKS:
            b = c % _NBUF
            if c >= _NBUF:
                wc[b].wait()
            gc[b] = pltpu.async_copy(table_hbm.at[idx_v.at[c]], rows[b], gsems[b])
        d = c - (_NBUF - 1)
        if d >= 0:
            b = d % _NBUF
            gc[b].wait()
            wc[b] = pltpu.async_copy(
                rows[b], out_hbm.at[pl.ds(base + d * _CHUNK, _CHUNK)], wsems[b]
            )
    for d in range(max(0, _N_CHUNKS - _NBUF), _N_CHUNKS):
        wc[d % _NBUF].wait()


@jax.jit
def _lookup(idx, table):
    mesh = plsc.VectorSubcoreMesh(
        core_axis_name="c", subcore_axis_name="s", num_cores=_NC, num_subcores=_NS
    )
    run = functools.partial(
        pl.kernel,
        out_type=jax.ShapeDtypeStruct((_B_PAD, _HIDDEN), jnp.float32),
        mesh=mesh,
        scratch_types=(
            [pltpu.VMEM((_N_CHUNKS, _CHUNK), jnp.int32)]
            + [pltpu.VMEM((_CHUNK, _HIDDEN), jnp.float32)] * _NBUF
            + [pltpu.SemaphoreType.DMA] * (2 * _NBUF)
        ),
    )(_body)
    return run(idx, table)


def kernel(x, emb_weight):
    n = x.shape[0]
    x = x.astype(jnp.int32)
    x_pad = jnp.pad(x, (0, _B_PAD - n)).reshape(_NW, _N_CHUNKS, _CHUNK)
    out = _lookup(x_pad, emb_weight.astype(jnp.float32))
    return out[:n]


# speedup vs baseline: 1.7044x; 1.6973x over previous
import functools

import jax
import jax.numpy as jnp
from jax import lax
from jax.experimental import pallas as pl
from jax.experimental.pallas import tpu as pltpu
from jax.experimental.pallas import tpu_sc as plsc

_HIDDEN = 128
_NC = 2
_NS = 16
_NW = _NC * _NS
_CHUNK = 128
_N_CHUNKS = 25
_B_PER_W = _N_CHUNKS * _CHUNK
_B_PAD = _NW * _B_PER_W


_NBUF = 4


def _body(idx_hbm, table_hbm, out_hbm, idx_v, *bufs):
    rows = bufs[:_NBUF]
    gsems = bufs[_NBUF : 2 * _NBUF]
    wsems = bufs[2 * _NBUF :]
    wid = lax.axis_index("s") * _NC + lax.axis_index("c")
    base = wid * _B_PER_W
    pltpu.sync_copy(idx_hbm.at[wid], idx_v)
    gc = [None] * _NBUF
    wc = [None] * _NBUF
    for c in range(_N_CHUNKS + _NBUF - 1):
        if c < _N_CHUNKS:
            b = c % _NBUF
            if c >= _NBUF:
                wc[b].wait()
            gc[b] = pltpu.async_copy(table_hbm.at[idx_v.at[c]], rows[b], gsems[b])
        d = c - (_NBUF - 1)
        if d >= 0:
            b = d % _NBUF
            gc[b].wait()
            wc[b] = pltpu.async_copy(
                rows[b], out_hbm.at[pl.ds(base + d * _CHUNK, _CHUNK)], wsems[b]
            )
    for d in range(max(0, _N_CHUNKS - _NBUF), _N_CHUNKS):
        wc[d % _NBUF].wait()


@jax.jit
def _lookup(idx, table):
    mesh = plsc.VectorSubcoreMesh(
        core_axis_name="c", subcore_axis_name="s", num_cores=_NC, num_subcores=_NS
    )
    run = functools.partial(
        pl.kernel,
        out_type=jax.ShapeDtypeStruct((_B_PAD, _HIDDEN), jnp.float32),
        mesh=mesh,
        scratch_types=(
            [pltpu.VMEM((_N_CHUNKS, _CHUNK), jnp.int32)]
            + [pltpu.VMEM((_CHUNK, _HIDDEN), jnp.float32)] * _NBUF
            + [pltpu.SemaphoreType.DMA] * (2 * _NBUF)
        ),
    )(_body)
    return run(idx, table)


def kernel(x, emb_weight):
    n = x.shape[0]
    num_emb = emb_weight.shape[0]
    x = x.astype(jnp.int32)
    x_pad = jnp.pad(x, (0, _B_PAD - n)).reshape(_NW, _N_CHUNKS, _CHUNK)
    x_pad = x_pad + (jnp.arange(_NW, dtype=jnp.int32) * num_emb)[:, None, None]
    table_rep = jnp.tile(emb_weight.astype(jnp.float32), (_NW, 1))
    out = _lookup(x_pad, table_rep)
    return out[:n]


# speedup vs baseline: 3.1106x; 1.8251x over previous
import functools

import jax
import jax.numpy as jnp
from jax import lax
from jax.experimental import pallas as pl
from jax.experimental.pallas import tpu as pltpu
from jax.experimental.pallas import tpu_sc as plsc

_HIDDEN = 128
_NC = 2
_NS = 16
_NW = _NC * _NS
_CHUNK = 128
_N_CHUNKS = 25
_B_PER_W = _N_CHUNKS * _CHUNK
_B_PAD = _NW * _B_PER_W


_NBUF = 4
_REP = 4


def _body(idx_hbm, table_hbm, out_hbm, idx_v, *bufs):
    rows = bufs[:_NBUF]
    gsems = bufs[_NBUF : 2 * _NBUF]
    wsems = bufs[2 * _NBUF :]
    wid = lax.axis_index("s") * _NC + lax.axis_index("c")
    base = wid * _B_PER_W
    pltpu.sync_copy(idx_hbm.at[wid], idx_v)
    gc = [None] * _NBUF
    wc = [None] * _NBUF
    for c in range(_N_CHUNKS + _NBUF - 1):
        if c < _N_CHUNKS:
            b = c % _NBUF
            if c >= _NBUF:
                wc[b].wait()
            gc[b] = pltpu.async_copy(table_hbm.at[idx_v.at[c]], rows[b], gsems[b])
        d = c - (_NBUF - 1)
        if d >= 0:
            b = d % _NBUF
            gc[b].wait()
            wc[b] = pltpu.async_copy(
                rows[b], out_hbm.at[pl.ds(base + d * _CHUNK, _CHUNK)], wsems[b]
            )
    for d in range(max(0, _N_CHUNKS - _NBUF), _N_CHUNKS):
        wc[d % _NBUF].wait()


@jax.jit
def _lookup(idx, table):
    mesh = plsc.VectorSubcoreMesh(
        core_axis_name="c", subcore_axis_name="s", num_cores=_NC, num_subcores=_NS
    )
    run = functools.partial(
        pl.kernel,
        out_type=jax.ShapeDtypeStruct((_B_PAD, _HIDDEN), jnp.float32),
        mesh=mesh,
        scratch_types=(
            [pltpu.VMEM((_N_CHUNKS, _CHUNK), jnp.int32)]
            + [pltpu.VMEM((_CHUNK, _HIDDEN), jnp.float32)] * _NBUF
            + [pltpu.SemaphoreType.DMA] * (2 * _NBUF)
        ),
    )(_body)
    return run(idx, table)


def kernel(x, emb_weight):
    n = x.shape[0]
    num_emb = emb_weight.shape[0]
    x = x.astype(jnp.int32)
    x_pad = jnp.pad(x, (0, _B_PAD - n)).reshape(_NW, _N_CHUNKS, _CHUNK)
    rep = (
        jnp.arange(_NW, dtype=jnp.int32)[:, None, None] * _REP
        + (jnp.arange(_CHUNK, dtype=jnp.int32) % _REP)[None, None, :]
    )
    x_pad = x_pad + rep * num_emb
    table_rep = jnp.tile(emb_weight.astype(jnp.float32), (_NW * _REP, 1))
    out = _lookup(x_pad, table_rep)
    return out[:n]


# speedup vs baseline: 3.6038x; 1.1585x over previous
import functools

import jax
import jax.numpy as jnp
from jax import lax
from jax.experimental import pallas as pl
from jax.experimental.pallas import tpu as pltpu
from jax.experimental.pallas import tpu_sc as plsc

_HIDDEN = 128
_NC = 2
_NS = 16
_NW = _NC * _NS
_CHUNK = 128
_N_CHUNKS = 25
_B_PER_W = _N_CHUNKS * _CHUNK
_B_PAD = _NW * _B_PER_W


_NBUF = 4
_REP = 8


def _body(idx_hbm, table_hbm, out_hbm, idx_v, *bufs):
    rows = bufs[:_NBUF]
    gsems = bufs[_NBUF : 2 * _NBUF]
    wsems = bufs[2 * _NBUF :]
    wid = lax.axis_index("s") * _NC + lax.axis_index("c")
    base = wid * _B_PER_W
    pltpu.sync_copy(idx_hbm.at[wid], idx_v)
    gc = [None] * _NBUF
    wc = [None] * _NBUF
    for c in range(_N_CHUNKS + _NBUF - 1):
        if c < _N_CHUNKS:
            b = c % _NBUF
            if c >= _NBUF:
                wc[b].wait()
            gc[b] = pltpu.async_copy(table_hbm.at[idx_v.at[c]], rows[b], gsems[b])
        d = c - (_NBUF - 1)
        if d >= 0:
            b = d % _NBUF
            gc[b].wait()
            wc[b] = pltpu.async_copy(
                rows[b], out_hbm.at[pl.ds(base + d * _CHUNK, _CHUNK)], wsems[b]
            )
    for d in range(max(0, _N_CHUNKS - _NBUF), _N_CHUNKS):
        wc[d % _NBUF].wait()


@jax.jit
def _lookup(idx, table):
    mesh = plsc.VectorSubcoreMesh(
        core_axis_name="c", subcore_axis_name="s", num_cores=_NC, num_subcores=_NS
    )
    run = functools.partial(
        pl.kernel,
        out_type=jax.ShapeDtypeStruct((_B_PAD, _HIDDEN), jnp.float32),
        mesh=mesh,
        scratch_types=(
            [pltpu.VMEM((_N_CHUNKS, _CHUNK), jnp.int32)]
            + [pltpu.VMEM((_CHUNK, _HIDDEN), jnp.float32)] * _NBUF
            + [pltpu.SemaphoreType.DMA] * (2 * _NBUF)
        ),
    )(_body)
    return run(idx, table)


def kernel(x, emb_weight):
    n = x.shape[0]
    num_emb = emb_weight.shape[0]
    x = x.astype(jnp.int32)
    x_pad = jnp.pad(x, (0, _B_PAD - n)).reshape(_NW, _N_CHUNKS, _CHUNK)
    rep = (
        jnp.arange(_NW, dtype=jnp.int32)[:, None, None] * _REP
        + (jnp.arange(_CHUNK, dtype=jnp.int32) % _REP)[None, None, :]
    )
    x_pad = x_pad + rep * num_emb
    table_rep = jnp.tile(emb_weight.astype(jnp.float32), (_NW * _REP, 1))
    out = _lookup(x_pad, table_rep)
    return out[:n]


# speedup vs baseline: 3.7246x; 1.0335x over previous
import functools

import jax
import jax.numpy as jnp
from jax import lax
from jax.experimental import pallas as pl
from jax.experimental.pallas import tpu as pltpu
from jax.experimental.pallas import tpu_sc as plsc

_HIDDEN = 128
_NC = 2
_NS = 16
_NW = _NC * _NS
_CHUNK = 128
_N_CHUNKS = 25
_B_PER_W = _N_CHUNKS * _CHUNK
_B_PAD = _NW * _B_PER_W


_NBUF = 4
_REP = 16


def _body(idx_hbm, table_hbm, out_hbm, idx_v, *bufs):
    rows = bufs[:_NBUF]
    gsems = bufs[_NBUF : 2 * _NBUF]
    wsems = bufs[2 * _NBUF :]
    wid = lax.axis_index("s") * _NC + lax.axis_index("c")
    base = wid * _B_PER_W
    pltpu.sync_copy(idx_hbm.at[wid], idx_v)
    gc = [None] * _NBUF
    wc = [None] * _NBUF
    for c in range(_N_CHUNKS + _NBUF - 1):
        if c < _N_CHUNKS:
            b = c % _NBUF
            if c >= _NBUF:
                wc[b].wait()
            gc[b] = pltpu.async_copy(table_hbm.at[idx_v.at[c]], rows[b], gsems[b])
        d = c - (_NBUF - 1)
        if d >= 0:
            b = d % _NBUF
            gc[b].wait()
            wc[b] = pltpu.async_copy(
                rows[b], out_hbm.at[pl.ds(base + d * _CHUNK, _CHUNK)], wsems[b]
            )
    for d in range(max(0, _N_CHUNKS - _NBUF), _N_CHUNKS):
        wc[d % _NBUF].wait()


@jax.jit
def _lookup(idx, table):
    mesh = plsc.VectorSubcoreMesh(
        core_axis_name="c", subcore_axis_name="s", num_cores=_NC, num_subcores=_NS
    )
    run = functools.partial(
        pl.kernel,
        out_type=jax.ShapeDtypeStruct((_B_PAD, _HIDDEN), jnp.float32),
        mesh=mesh,
        scratch_types=(
            [pltpu.VMEM((_N_CHUNKS, _CHUNK), jnp.int32)]
            + [pltpu.VMEM((_CHUNK, _HIDDEN), jnp.float32)] * _NBUF
            + [pltpu.SemaphoreType.DMA] * (2 * _NBUF)
        ),
    )(_body)
    return run(idx, table)


def kernel(x, emb_weight):
    n = x.shape[0]
    num_emb = emb_weight.shape[0]
    x = x.astype(jnp.int32)
    x_pad = jnp.pad(x, (0, _B_PAD - n)).reshape(_NW, _N_CHUNKS, _CHUNK)
    rep = (
        jnp.arange(_NW, dtype=jnp.int32)[:, None, None] * _REP
        + (jnp.arange(_CHUNK, dtype=jnp.int32) % _REP)[None, None, :]
    )
    x_pad = x_pad + rep * num_emb
    table_rep = jnp.tile(emb_weight.astype(jnp.float32), (_NW * _REP, 1))
    out = _lookup(x_pad, table_rep)
    return out[:n]


# speedup vs baseline: 6.0811x; 1.6327x over previous
import functools

import jax
import jax.numpy as jnp
from jax import lax
from jax.experimental import pallas as pl
from jax.experimental.pallas import tpu as pltpu
from jax.experimental.pallas import tpu_sc as plsc

_N = 100000
_HIDDEN = 128
_NC = 2
_NS = 16
_NW = _NC * _NS
_CHUNK = 128
_BIG = 3128
_SMALL = 3120
_N_BIG = 20
_N_FULL = 24
_TAIL_BIG = _BIG - _N_FULL * _CHUNK
_TAIL_SMALL = _SMALL - _N_FULL * _CHUNK
_NBUF = 4
_REP = 16


def _pipeline(table_hbm, out_hbm, idx_v, rows, gsems, wsems, base, tail):
    n_chunks = _N_FULL + 1
    gc = [None] * _NBUF
    wc = [None] * _NBUF
    for c in range(n_chunks + _NBUF - 1):
        if c < n_chunks:
            b = c % _NBUF
            cnt = _CHUNK if c < _N_FULL else tail
            if c >= _NBUF:
                wc[b].wait()
            gc[b] = pltpu.async_copy(
                table_hbm.at[idx_v.at[pl.ds(c * _CHUNK, cnt)]],
                rows[b].at[pl.ds(0, cnt)],
                gsems[b],
            )
        d = c - (_NBUF - 1)
        if d >= 0:
            b = d % _NBUF
            cnt = _CHUNK if d < _N_FULL else tail
            gc[b].wait()
            wc[b] = pltpu.async_copy(
                rows[b].at[pl.ds(0, cnt)],
                out_hbm.at[pl.ds(base + d * _CHUNK, cnt)],
                wsems[b],
            )
    for d in range(max(0, n_chunks - _NBUF), n_chunks):
        wc[d % _NBUF].wait()


def _body(idx_hbm, table_hbm, out_hbm, idx_v, *bufs):
    rows = bufs[:_NBUF]
    gsems = bufs[_NBUF : 2 * _NBUF]
    wsems = bufs[2 * _NBUF :]
    wid = lax.axis_index("s") * _NC + lax.axis_index("c")
    is_big = wid < _N_BIG
    base = jnp.where(
        is_big, wid * _BIG, _N_BIG * _BIG + (wid - _N_BIG) * _SMALL
    )
    base = pl.multiple_of(base, 8)

    @pl.when(is_big)
    def _():
        pltpu.sync_copy(idx_hbm.at[pl.ds(base, _BIG)], idx_v)
        _pipeline(table_hbm, out_hbm, idx_v, rows, gsems, wsems, base, _TAIL_BIG)

    @pl.when(jnp.logical_not(is_big))
    def _():
        pltpu.sync_copy(
            idx_hbm.at[pl.ds(base, _SMALL)], idx_v.at[pl.ds(0, _SMALL)]
        )
        _pipeline(table_hbm, out_hbm, idx_v, rows, gsems, wsems, base, _TAIL_SMALL)


@jax.jit
def _lookup(idx, table):
    mesh = plsc.VectorSubcoreMesh(
        core_axis_name="c", subcore_axis_name="s", num_cores=_NC, num_subcores=_NS
    )
    run = functools.partial(
        pl.kernel,
        out_type=jax.ShapeDtypeStruct((_N, _HIDDEN), jnp.float32),
        mesh=mesh,
        scratch_types=(
            [pltpu.VMEM((_BIG,), jnp.int32)]
            + [pltpu.VMEM((_CHUNK, _HIDDEN), jnp.float32)] * _NBUF
            + [pltpu.SemaphoreType.DMA] * (2 * _NBUF)
        ),
    )(_body)
    return run(idx, table)


def kernel(x, emb_weight):
    num_emb = emb_weight.shape[0]
    x = x.astype(jnp.int32)
    wid_of_pos = jnp.minimum(
        jnp.arange(_N, dtype=jnp.int32) // _SMALL, _NW - 1
    )
    rep = wid_of_pos * _REP + (jnp.arange(_N, dtype=jnp.int32) % _REP)
    xr = x + rep * num_emb
    table_rep = jnp.tile(emb_weight.astype(jnp.float32), (_NW * _REP + _REP, 1))
    return _lookup(xr, table_rep)


# speedup vs baseline: 6.5969x; 1.0848x over previous
import functools

import jax
import jax.numpy as jnp
from jax import lax
from jax.experimental import pallas as pl
from jax.experimental.pallas import tpu as pltpu
from jax.experimental.pallas import tpu_sc as plsc

_N = 100000
_HIDDEN = 128
_NC = 2
_NS = 16
_NW = _NC * _NS
_CHUNK = 128
_BIG = 3128
_SMALL = 3120
_N_BIG = 20
_N_FULL = 24
_TAIL_BIG = _BIG - _N_FULL * _CHUNK
_TAIL_SMALL = _SMALL - _N_FULL * _CHUNK
_NBUF = 6
_REP = 8


def _pipeline(table_hbm, out_hbm, idx_v, rows, gsems, wsems, base, tail):
    n_chunks = _N_FULL + 1
    gc = [None] * _NBUF
    wc = [None] * _NBUF
    for c in range(n_chunks + _NBUF - 1):
        if c < n_chunks:
            b = c % _NBUF
            cnt = _CHUNK if c < _N_FULL else tail
            if c >= _NBUF:
                wc[b].wait()
            gc[b] = pltpu.async_copy(
                table_hbm.at[idx_v.at[pl.ds(c * _CHUNK, cnt)]],
                rows[b].at[pl.ds(0, cnt)],
                gsems[b],
            )
        d = c - (_NBUF - 1)
        if d >= 0:
            b = d % _NBUF
            cnt = _CHUNK if d < _N_FULL else tail
            gc[b].wait()
            wc[b] = pltpu.async_copy(
                rows[b].at[pl.ds(0, cnt)],
                out_hbm.at[pl.ds(base + d * _CHUNK, cnt)],
                wsems[b],
            )
    for d in range(max(0, n_chunks - _NBUF), n_chunks):
        wc[d % _NBUF].wait()


def _body(idx_hbm, table_hbm, out_hbm, idx_v, *bufs):
    rows = bufs[:_NBUF]
    gsems = bufs[_NBUF : 2 * _NBUF]
    wsems = bufs[2 * _NBUF :]
    wid = lax.axis_index("s") * _NC + lax.axis_index("c")
    is_big = wid < _N_BIG
    base = jnp.where(
        is_big, wid * _BIG, _N_BIG * _BIG + (wid - _N_BIG) * _SMALL
    )
    base = pl.multiple_of(base, 8)

    @pl.when(is_big)
    def _():
        pltpu.sync_copy(idx_hbm.at[pl.ds(base, _BIG)], idx_v)
        _pipeline(table_hbm, out_hbm, idx_v, rows, gsems, wsems, base, _TAIL_BIG)

    @pl.when(jnp.logical_not(is_big))
    def _():
        pltpu.sync_copy(
            idx_hbm.at[pl.ds(base, _SMALL)], idx_v.at[pl.ds(0, _SMALL)]
        )
        _pipeline(table_hbm, out_hbm, idx_v, rows, gsems, wsems, base, _TAIL_SMALL)


@jax.jit
def _lookup(idx, table):
    mesh = plsc.VectorSubcoreMesh(
        core_axis_name="c", subcore_axis_name="s", num_cores=_NC, num_subcores=_NS
    )
    run = functools.partial(
        pl.kernel,
        out_type=jax.ShapeDtypeStruct((_N, _HIDDEN), jnp.float32),
        mesh=mesh,
        scratch_types=(
            [pltpu.VMEM((_BIG,), jnp.int32)]
            + [pltpu.VMEM((_CHUNK, _HIDDEN), jnp.float32)] * _NBUF
            + [pltpu.SemaphoreType.DMA] * (2 * _NBUF)
        ),
    )(_body)
    return run(idx, table)


def kernel(x, emb_weight):
    num_emb = emb_weight.shape[0]
    x = x.astype(jnp.int32)
    wid_of_pos = jnp.minimum(
        jnp.arange(_N, dtype=jnp.int32) // _SMALL, _NW - 1
    )
    rep = wid_of_pos * _REP + (jnp.arange(_N, dtype=jnp.int32) % _REP)
    xr = x + rep * num_emb
    table_rep = jnp.tile(emb_weight.astype(jnp.float32), (_NW * _REP + _REP, 1))
    return _lookup(xr, table_rep)
